# baseline (speedup 1.0000x reference)
import functools
import math

import jax
import jax.numpy as jnp
from jax import lax
from jax.experimental import pallas as pl
from jax.experimental.pallas import tpu as pltpu
from jax.experimental.pallas import tpu_sc as plsc

B = 128
N = 32768
K = math.ceil(0.25 * B)
L = 16
NC = 2
NS = 16
NW = NC * NS
COLS_PER_W = N // NW
CB = 512
DEPTH = 4
H = CB // DEPTH
NGRP = H // L
NCH = B // K


def _oddeven_merge(lo, hi, r):
    step = r * 2
    if step < hi - lo:
        yield from _oddeven_merge(lo, hi, step)
        yield from _oddeven_merge(lo + r, hi, step)
        for i in range(lo + r, hi - r, step):
            yield (i, i + r)
    else:
        yield (lo, lo + r)


def _oddeven_sort_pairs(lo, hi):
    if (hi - lo) >= 1:
        mid = lo + ((hi - lo) // 2)
        yield from _oddeven_sort_pairs(lo, mid)
        yield from _oddeven_sort_pairs(mid + 1, hi)
        yield from _oddeven_merge(lo, hi, 1)


_SORT32_PAIRS = tuple(_oddeven_sort_pairs(0, 31))


def _sort32(a):
    for i, l in _SORT32_PAIRS:
        lo = jnp.minimum(a[i], a[l])
        hi = jnp.maximum(a[i], a[l])
        a[i], a[l] = lo, hi


def _merge_top32(s, c):
    m = [jnp.maximum(s[i], c[31 - i]) for i in range(32)]
    for j in (16, 8, 4, 2, 1):
        for i in range(32):
            l = i ^ j
            if l > i:
                lo = jnp.minimum(m[i], m[l])
                hi = jnp.maximum(m[i], m[l])
                m[i], m[l] = lo, hi
    return m


NPH = COLS_PER_W // H


def _sc_body(x_hbm, out_hbm, buf, si, so):
    wid = lax.axis_index("s") * NC + lax.axis_index("c")

    def in_half(p):
        c0 = wid * COLS_PER_W + p * H
        return pltpu.async_copy(
            x_hbm.at[:, pl.ds(c0, H)], buf.at[:, pl.ds((p % DEPTH) * H, H)],
            si)

    def out_half(p):
        c0 = wid * COLS_PER_W + p * H
        return pltpu.async_copy(
            buf.at[:, pl.ds((p % DEPTH) * H, H)], out_hbm.at[:, pl.ds(c0, H)],
            so)

    def wait_half(sem):
        pltpu.make_async_copy(
            x_hbm.at[:, pl.ds(0, H)], buf.at[:, pl.ds(0, H)], sem).wait()

    def compute_half(base):
        @plsc.parallel_loop(0, NGRP, unroll=2)
        def grp_body(g):
            goff = base + g * L

            def ldr(r):
                return buf[r, pl.ds(goff, L)]

            s = [ldr(i) for i in range(K)]
            _sort32(s)
            for ch in range(1, NCH - 1):
                c = [ldr(ch * K + i) for i in range(K)]
                _sort32(c)
                s = _merge_top32(s, c)
            c = [ldr((NCH - 1) * K + i) for i in range(K)]
            _sort32(c)
            m = [jnp.maximum(s[i], c[31 - i]) for i in range(K)]
            mins = m
            while len(mins) > 1:
                mins = [jnp.minimum(mins[2 * i], mins[2 * i + 1])
                        for i in range(len(mins) // 2)]
            t = mins[0]
            rem = (m[0] == t).astype(jnp.int32)
            for i in range(1, K):
                rem = rem + (m[i] == t).astype(jnp.int32)
            eq_seen = jnp.zeros((L,), jnp.int32)
            for r in range(B):
                v = ldr(r)
                gt = v > t
                eq = v == t
                keep = gt | (eq & (eq_seen < rem))
                eq_seen = eq_seen + eq.astype(jnp.int32)
                buf[r, pl.ds(goff, L)] = jnp.where(keep, v, jnp.float32(0.0))

    for p in range(DEPTH):
        in_half(p)

    def phase(p, carry):
        wait_half(si)
        compute_half((p % DEPTH) * H)
        out_half(p)

        @pl.when(p + DEPTH < NPH)
        def _():
            wait_half(so)
            in_half(p + DEPTH)

        return carry

    lax.fori_loop(0, NPH, phase, jnp.int32(0))
    for _ in range(DEPTH):
        wait_half(so)


_mesh = plsc.VectorSubcoreMesh(core_axis_name="c", subcore_axis_name="s")


@jax.jit
def kernel(x):
    f = pl.kernel(
        _sc_body,
        out_type=jax.ShapeDtypeStruct((B, N), jnp.float32),
        mesh=_mesh,
        scratch_types=[pltpu.VMEM((B, CB), jnp.float32),
                       pltpu.SemaphoreType.DMA,
                       pltpu.SemaphoreType.DMA],
        compiler_params=pltpu.CompilerParams(needs_layout_passes=False),
    )
    return f(x)


# speedup vs baseline: 2.0477x; 2.0477x over previous
import math

import jax
import jax.numpy as jnp
from jax import lax
from jax.experimental import pallas as pl
from jax.experimental.pallas import tpu as pltpu
from jax.experimental.pallas import tpu_sc as plsc

B = 128
N = 32768
K = math.ceil(0.25 * B)
L = 16
NC = 2
NS = 16
NW = NC * NS
COLS_PER_W = N // NW
CB = 512
DEPTH = 4
H = CB // DEPTH
NGRP = H // L
NCH = B // K


def _oddeven_merge(lo, hi, r):
    step = r * 2
    if step < hi - lo:
        yield from _oddeven_merge(lo, hi, step)
        yield from _oddeven_merge(lo + r, hi, step)
        for i in range(lo + r, hi - r, step):
            yield (i, i + r)
    else:
        yield (lo, lo + r)


def _oddeven_sort_pairs(lo, hi):
    if (hi - lo) >= 1:
        mid = lo + ((hi - lo) // 2)
        yield from _oddeven_sort_pairs(lo, mid)
        yield from _oddeven_sort_pairs(mid + 1, hi)
        yield from _oddeven_merge(lo, hi, 1)


_SORT32_PAIRS = tuple(_oddeven_sort_pairs(0, 31))


def _sort32(a):
    for i, l in _SORT32_PAIRS:
        lo = jnp.minimum(a[i], a[l])
        hi = jnp.maximum(a[i], a[l])
        a[i], a[l] = lo, hi


def _merge_top32(s, c):
    m = [jnp.maximum(s[i], c[31 - i]) for i in range(32)]
    for j in (16, 8, 4, 2, 1):
        for i in range(32):
            l = i ^ j
            if l > i:
                lo = jnp.minimum(m[i], m[l])
                hi = jnp.maximum(m[i], m[l])
                m[i], m[l] = lo, hi
    return m


NPH = COLS_PER_W // H


def _sc_body(x_hbm, out_hbm, buf, si, so):
    wid = lax.axis_index("s") * NC + lax.axis_index("c")

    def in_half(p):
        c0 = wid * COLS_PER_W + p * H
        return pltpu.async_copy(
            x_hbm.at[:, pl.ds(c0, H)], buf.at[:, pl.ds((p % DEPTH) * H, H)],
            si)

    def out_half(p):
        c0 = wid * COLS_PER_W + p * H
        return pltpu.async_copy(
            buf.at[:, pl.ds((p % DEPTH) * H, H)], out_hbm.at[:, pl.ds(c0, H)],
            so)

    def wait_half(sem):
        pltpu.make_async_copy(
            x_hbm.at[:, pl.ds(0, H)], buf.at[:, pl.ds(0, H)], sem).wait()

    def compute_half(base):
        @plsc.parallel_loop(0, NGRP, unroll=1)
        def grp_body(g):
            goff = base + g * L

            def ldr(r):
                return buf[r, pl.ds(goff, L)]

            s = [ldr(i) for i in range(K)]
            _sort32(s)
            for ch in range(1, NCH - 1):
                c = [ldr(ch * K + i) for i in range(K)]
                _sort32(c)
                s = _merge_top32(s, c)
            c = [ldr((NCH - 1) * K + i) for i in range(K)]
            _sort32(c)
            m = [jnp.maximum(s[i], c[31 - i]) for i in range(K)]
            mins = m
            while len(mins) > 1:
                mins = [jnp.minimum(mins[2 * i], mins[2 * i + 1])
                        for i in range(len(mins) // 2)]
            t = mins[0]
            rem = (m[0] == t).astype(jnp.int32)
            for i in range(1, K):
                rem = rem + (m[i] == t).astype(jnp.int32)
            eq_seen = jnp.zeros((L,), jnp.int32)
            for r in range(B):
                v = ldr(r)
                gt = v > t
                eq = v == t
                keep = gt | (eq & (eq_seen < rem))
                eq_seen = eq_seen + eq.astype(jnp.int32)
                buf[r, pl.ds(goff, L)] = jnp.where(keep, v, jnp.float32(0.0))

    for p in range(DEPTH):
        in_half(p)

    def phase(p, carry):
        wait_half(si)
        compute_half((p % DEPTH) * H)
        out_half(p)

        @pl.when(p + DEPTH < NPH)
        def _():
            wait_half(so)
            in_half(p + DEPTH)

        return carry

    lax.fori_loop(0, NPH, phase, jnp.int32(0))
    for _ in range(DEPTH):
        wait_half(so)


_mesh = plsc.VectorSubcoreMesh(core_axis_name="c", subcore_axis_name="s")


@jax.jit
def kernel(x):
    f = pl.kernel(
        _sc_body,
        out_type=jax.ShapeDtypeStruct((B, N), jnp.float32),
        mesh=_mesh,
        scratch_types=[pltpu.VMEM((B, CB), jnp.float32),
                       pltpu.SemaphoreType.DMA,
                       pltpu.SemaphoreType.DMA],
        compiler_params=pltpu.CompilerParams(needs_layout_passes=False),
    )
    return f(x)
